# Initial kernel scaffold; baseline (speedup 1.0000x reference)
import jax
import jax.numpy as jnp
from jax.experimental import pallas as pl


def kernel(x, pos, batch, params):
    raise NotImplementedError("write your pallas kernel here")


# speedup vs baseline: 4.2374x; 4.2374x over previous
import functools

import jax
import jax.numpy as jnp
from jax.experimental import pallas as pl
from jax.experimental.pallas import tpu as pltpu
from jax.experimental.pallas import tpu_sc as plsc

_K = 16
_B = 16


def _pick_div(n, target):
    for r in range(target, 7, -1):
        if n % r == 0 and r % 8 == 0:
            return r
    for r in range(target, 0, -1):
        if n % r == 0:
            return r
    return n


def _knn_body(bounds_ref, pos16_ref, posT_ref, pos16p_ref, nbr_ref, eattr_ref,
              dbuf, *, R, C, K, N):
    i = pl.program_id(0)
    lo = bounds_ref[i, 0]
    hi = bounds_ref[i, 1]
    nt = (hi - lo + C - 1) // C

    p0 = pos16_ref[:, 0:1]
    p1 = pos16_ref[:, 1:2]
    p2 = pos16_ref[:, 2:3]
    bi = pos16_ref[:, 3:4]
    sqi = (p0 * p0 + p1 * p1) + p2 * p2

    def dist_tile(t, _):
        c0 = pl.multiple_of(lo + t * C, 512)
        cl = pl.multiple_of(t * C, 512)
        q0 = posT_ref[0:1, pl.ds(c0, C)]
        q1 = posT_ref[1:2, pl.ds(c0, C)]
        q2 = posT_ref[2:3, pl.ds(c0, C)]
        bj = posT_ref[3:4, pl.ds(c0, C)]
        sqj = (q0 * q0 + q1 * q1) + q2 * q2
        dot = (p0 * q0 + p1 * q1) + p2 * q2
        d2 = (sqi + sqj) - 2.0 * dot
        d2 = jnp.where(bj != bi, jnp.inf, d2)
        dbuf[:, pl.ds(cl, C)] = d2
        return 0

    jax.lax.fori_loop(0, nt, dist_tile, 0)

    big_i = jnp.int32(2 ** 30)
    for it in range(K):
        def min_tile(t, m):
            tile = dbuf[:, pl.ds(pl.multiple_of(t * C, 512), C)]
            return jnp.minimum(m, jnp.min(tile, axis=1, keepdims=True))
        m = jax.lax.fori_loop(0, nt, min_tile,
                              jnp.full((R, 1), jnp.inf, jnp.float32))

        def idx_tile(t, cur):
            tile = dbuf[:, pl.ds(pl.multiple_of(t * C, 512), C)]
            ii = jax.lax.broadcasted_iota(jnp.int32, (R, C), 1) + t * C
            cand = jnp.where(tile == m, ii, big_i)
            return jnp.minimum(cur, jnp.min(cand, axis=1, keepdims=True))
        idx = jax.lax.fori_loop(0, nt, idx_tile,
                                jnp.full((R, 1), big_i, jnp.int32))

        def mask_tile(t, _):
            cl = pl.multiple_of(t * C, 512)
            tile = dbuf[:, pl.ds(cl, C)]
            ii = jax.lax.broadcasted_iota(jnp.int32, (R, C), 1) + t * C
            dbuf[:, pl.ds(cl, C)] = jnp.where(ii == idx, jnp.inf, tile)
            return 0
        jax.lax.fori_loop(0, nt, mask_tile, 0)

        gidx = jnp.minimum(lo + jnp.where(idx >= big_i, 0, idx), N - 1)
        nbr_ref[:, it:it + 1] = gidx

        def pj_tile(t, pj):
            ii = jax.lax.broadcasted_iota(jnp.int32, (R, C), 1) + t * C
            oh = (ii == idx).astype(jnp.float32)
            win = pos16p_ref[pl.ds(pl.multiple_of(lo + t * C, 512), C), 0:8]
            return pj + jnp.dot(oh, win, preferred_element_type=jnp.float32)
        pj = jax.lax.fori_loop(0, nt, pj_tile,
                               jnp.zeros((R, 8), jnp.float32))
        diff = pj - pos16_ref[:, 0:8]
        d2e = jnp.sum(diff * diff, axis=1, keepdims=True)
        dist = jnp.sqrt(d2e + 1e-12)
        dirv = diff * (1.0 / (dist + 1e-6))
        eattr_ref[it] = jnp.concatenate([dist, dirv[:, 0:7]], axis=1)


def _knn(pos16, posT8, pos16p, bounds, N, R, C, K):
    CMAX = ((N + C - 1) // C) * C
    NPAD = posT8.shape[1]
    return pl.pallas_call(
        functools.partial(_knn_body, R=R, C=C, K=K, N=N),
        grid_spec=pltpu.PrefetchScalarGridSpec(
            num_scalar_prefetch=1,
            grid=(N // R,),
            in_specs=[
                pl.BlockSpec((R, 16), lambda i, s: (i, 0)),
                pl.BlockSpec((8, NPAD), lambda i, s: (0, 0)),
                pl.BlockSpec((NPAD, 16), lambda i, s: (0, 0)),
            ],
            out_specs=[
                pl.BlockSpec((R, K), lambda i, s: (i, 0)),
                pl.BlockSpec((K, R, 8), lambda i, s: (0, i, 0)),
            ],
            scratch_shapes=[pltpu.VMEM((R, CMAX), jnp.float32)],
        ),
        out_shape=[
            jax.ShapeDtypeStruct((N, K), jnp.int32),
            jax.ShapeDtypeStruct((K, N, 8), jnp.float32),
        ],
    )(bounds, pos16, posT8, pos16p)


def _init_body(x_ref, Win, bin_, We1a, be1, h_ref, g_ref):
    h = jnp.dot(x_ref[...], Win[...],
                preferred_element_type=jnp.float32) + bin_[...]
    h_ref[...] = h
    g_ref[...] = jnp.dot(h, We1a[...],
                         preferred_element_type=jnp.float32) + be1[...]


def _init(x, Win, bin_, We1a, be1, N, R, H):
    D = x.shape[1]
    return pl.pallas_call(
        _init_body,
        grid=(N // R,),
        in_specs=[
            pl.BlockSpec((R, D), lambda i: (i, 0)),
            pl.BlockSpec((D, H), lambda i: (0, 0)),
            pl.BlockSpec((1, H), lambda i: (0, 0)),
            pl.BlockSpec((H, H), lambda i: (0, 0)),
            pl.BlockSpec((1, H), lambda i: (0, 0)),
        ],
        out_specs=[
            pl.BlockSpec((R, H), lambda i: (i, 0)),
            pl.BlockSpec((R, H), lambda i: (i, 0)),
        ],
        out_shape=[
            jax.ShapeDtypeStruct((N, H), jnp.float32),
            jax.ShapeDtypeStruct((N, H), jnp.float32),
        ],
    )(x, Win, bin_, We1a, be1)


def _sc_gather(table, idx_flat):
    E = idx_flat.shape[0]
    W = table.shape[1]
    win = 128
    grid = E // win
    mesh = plsc.VectorSubcoreMesh(core_axis_name="c", subcore_axis_name="s")

    @functools.partial(
        pl.kernel,
        out_type=jax.ShapeDtypeStruct((E, W), table.dtype),
        mesh=mesh)
    def k(x_hbm, i_hbm, o_hbm):
        def body(i_vmem, o_vmem):
            pltpu.sync_copy(x_hbm.at[i_vmem.at[0]], o_vmem)

        pltpu.emit_pipeline(
            body,
            grid=(grid,),
            in_specs=[pl.BlockSpec((1, win), index_map=lambda i: (0, i))],
            out_specs=[pl.BlockSpec((win, W), index_map=lambda i: (i, 0))],
            core_axis_name=("c", "s"),
            dimension_semantics=(pltpu.PARALLEL,),
        )(i_hbm, o_hbm)

    return k(table, idx_flat.reshape(1, E))


def _layer_body(h_ref, gg_ref, ea_ref, We1b, We1c8, be2, We2,
                Wh1a, Wh1b, bh1, Wh2, bh2, *args, R, K, H, has_next):
    if has_next:
        We1an, be1n, hout_ref, gout_ref = args
    else:
        (hout_ref,) = args

    h = h_ref[...]
    c = jnp.dot(h, We1b[...], preferred_element_type=jnp.float32)
    repC = jnp.repeat(c, K, axis=0)
    gg = gg_ref[...]
    eterm = jnp.dot(ea_ref[...], We1c8[...],
                    preferred_element_type=jnp.float32)

    m1 = gg + repC + eterm
    m1 = m1 * jax.nn.sigmoid(m1)
    m2 = jnp.dot(m1, We2[...], preferred_element_type=jnp.float32) + be2[...]
    m2 = m2 * jax.nn.sigmoid(m2)
    agg = jnp.sum(m2.reshape(R, K, H), axis=1)

    u = (jnp.dot(h, Wh1a[...], preferred_element_type=jnp.float32)
         + jnp.dot(agg, Wh1b[...], preferred_element_type=jnp.float32)
         + bh1[...])
    u = u * jax.nn.sigmoid(u)
    u = jnp.dot(u, Wh2[...], preferred_element_type=jnp.float32) + bh2[...]
    hn = h + u
    hout_ref[...] = hn
    if has_next:
        gout_ref[...] = jnp.dot(hn, We1an[...],
                                preferred_element_type=jnp.float32) + be1n[...]


def _layer(h, gg, eattr8, wts, nxt, N, R, K, H):
    has_next = nxt is not None
    whole = lambda shape: pl.BlockSpec(shape, lambda i: (0, 0))
    in_specs = [
        pl.BlockSpec((R, H), lambda i: (i, 0)),
        pl.BlockSpec((R * K, H), lambda i: (i, 0)),
        pl.BlockSpec((R * K, 8), lambda i: (i, 0)),
        whole((H, H)), whole((8, H)), whole((1, H)), whole((H, H)),
        whole((H, H)), whole((H, H)), whole((1, H)), whole((H, H)),
        whole((1, H)),
    ]
    operands = [h, gg, eattr8] + list(wts)
    out_specs = [pl.BlockSpec((R, H), lambda i: (i, 0))]
    out_shape = [jax.ShapeDtypeStruct((N, H), jnp.float32)]
    if has_next:
        in_specs += [whole((H, H)), whole((1, H))]
        operands += list(nxt)
        out_specs.append(pl.BlockSpec((R, H), lambda i: (i, 0)))
        out_shape.append(jax.ShapeDtypeStruct((N, H), jnp.float32))
    return pl.pallas_call(
        functools.partial(_layer_body, R=R, K=K, H=H, has_next=has_next),
        grid=(N // R,),
        in_specs=in_specs,
        out_specs=out_specs,
        out_shape=out_shape,
    )(*operands)


def _pool_body(h_ref, pos16_ref, Wo1, bo1, Wo2, bo2, Wo3T, bo3, out_ref,
               acc, cnt, ones_sc, *, R, B, nblk):
    i = pl.program_id(0)

    @pl.when(i == 0)
    def _():
        acc[...] = jnp.zeros_like(acc)
        cnt[...] = jnp.zeros_like(cnt)
        ones_sc[...] = jnp.ones_like(ones_sc)

    b = pos16_ref[:, 3:4].astype(jnp.int32)
    iota_b = jax.lax.broadcasted_iota(jnp.int32, (R, B), 1)
    oh = (b == iota_b).astype(jnp.float32)
    dn = (((0,), (0,)), ((), ()))
    acc[...] += jax.lax.dot_general(oh, h_ref[...], dn,
                                    preferred_element_type=jnp.float32)
    cnt[...] += jax.lax.dot_general(oh, ones_sc[...], dn,
                                    preferred_element_type=jnp.float32)

    @pl.when(i == nblk - 1)
    def _():
        pooled = acc[...] / jnp.maximum(cnt[...], 1.0)
        o = jnp.maximum(jnp.dot(pooled, Wo1[...],
                                preferred_element_type=jnp.float32)
                        + bo1[...], 0.0)
        o = jnp.maximum(jnp.dot(o, Wo2[...],
                                preferred_element_type=jnp.float32)
                        + bo2[...], 0.0)
        out_ref[...] = jnp.sum(o * Wo3T[...], axis=1,
                               keepdims=True) + bo3[...]


def _pool(h, pos16, Wo1, bo1, Wo2, bo2, Wo3T, bo3, N, R, B, H):
    whole = lambda shape: pl.BlockSpec(shape, lambda i: (0, 0))
    nblk = N // R
    return pl.pallas_call(
        functools.partial(_pool_body, R=R, B=B, nblk=nblk),
        grid=(nblk,),
        in_specs=[
            pl.BlockSpec((R, H), lambda i: (i, 0)),
            pl.BlockSpec((R, 16), lambda i: (i, 0)),
            whole((H, 2 * H)), whole((1, 2 * H)),
            whole((2 * H, H)), whole((1, H)),
            whole((1, H)), whole((1, 1)),
        ],
        out_specs=pl.BlockSpec((B, 1), lambda i: (0, 0)),
        out_shape=jax.ShapeDtypeStruct((B, 1), jnp.float32),
        scratch_shapes=[
            pltpu.VMEM((B, H), jnp.float32),
            pltpu.VMEM((B, 1), jnp.float32),
            pltpu.VMEM((R, 1), jnp.float32),
        ],
    )(h, pos16, Wo1, bo1, Wo2, bo2, Wo3T, bo3)


def kernel(x, pos, batch, params):
    N, D = x.shape
    H = params['W_in'].shape[1]
    K, B = _K, _B
    C = 512
    R = _pick_div(N, 80)

    batchf = batch.astype(jnp.float32)
    pos16 = jnp.concatenate(
        [pos, batchf[:, None], jnp.zeros((N, 16 - pos.shape[1] - 1),
                                         jnp.float32)], axis=1)

    NPAD = (((N + C - 1) // C) + 1) * C
    pt = jnp.concatenate([pos.T, batchf[None, :]], axis=0)
    pad = jnp.concatenate(
        [jnp.zeros((3, NPAD - N), jnp.float32),
         -jnp.ones((1, NPAD - N), jnp.float32)], axis=0)
    posT8 = jnp.concatenate(
        [jnp.concatenate([pt, pad], axis=1),
         jnp.zeros((4, NPAD), jnp.float32)], axis=0)

    pos16p = jnp.concatenate(
        [pos16, jnp.zeros((NPAD - N, 16), jnp.float32)], axis=0)

    ar = jnp.arange(B)
    seg_start = jnp.searchsorted(batch, ar, side='left').astype(jnp.int32)
    seg_end = jnp.searchsorted(batch, ar, side='right').astype(jnp.int32)
    firstb = batch[::R]
    lastb = batch[R - 1::R]
    bounds = jnp.stack([(seg_start[firstb] // C) * C, seg_end[lastb]], axis=1)

    nbr, eattr = _knn(pos16, posT8, pos16p, bounds, N, R, C, K)
    idx_flat = nbr.reshape(-1)
    eattr8 = eattr.transpose(1, 0, 2).reshape(N * K, 8)

    layers = params['layers']
    w0 = layers[0]
    b2 = lambda v: v.reshape(1, -1)
    We1a0 = w0['We1'][:H]
    be10 = b2(w0['be1'])
    h, g = _init(x, params['W_in'], b2(params['b_in']),
                 We1a0, be10, N, R, H)

    for li, p in enumerate(layers):
        gg = _sc_gather(g, idx_flat)
        We1c8 = jnp.concatenate(
            [p['We1'][2 * H:], jnp.zeros((8 - (p['We1'].shape[0] - 2 * H), H),
                                         jnp.float32)], axis=0)
        wts = (p['We1'][H:2 * H], We1c8, b2(p['be2']), p['We2'],
               p['Wh1'][:H], p['Wh1'][H:], b2(p['bh1']), p['Wh2'],
               b2(p['bh2']))
        if li + 1 < len(layers):
            pn = layers[li + 1]
            nxt = (pn['We1'][:H], b2(pn['be1']))
            h, g = _layer(h, gg, eattr8, wts, nxt, N, R, K, H)
        else:
            (h,) = _layer(h, gg, eattr8, wts, None, N, R, K, H)

    return _pool(h, pos16, params['Wo1'], b2(params['bo1']),
                 params['Wo2'], b2(params['bo2']),
                 params['Wo3'].T, b2(params['bo3']), N, R, B, H)


# speedup vs baseline: 5.1114x; 1.2063x over previous
import functools

import jax
import jax.numpy as jnp
from jax.experimental import pallas as pl
from jax.experimental.pallas import tpu as pltpu
from jax.experimental.pallas import tpu_sc as plsc

_K = 16
_B = 16


def _pick_div(n, target):
    for r in range(target, 7, -1):
        if n % r == 0 and r % 8 == 0:
            return r
    for r in range(target, 0, -1):
        if n % r == 0:
            return r
    return n


def _knn_body(bounds_ref, pos16_ref, posT_ref, pos16p_ref, nbr_ref, eattr_ref,
              dbuf, *, R, C, K, N):
    i = pl.program_id(0)
    lo = bounds_ref[i, 0]
    hi = bounds_ref[i, 1]
    nt = (hi - lo + C - 1) // C

    p0 = pos16_ref[:, 0:1]
    p1 = pos16_ref[:, 1:2]
    p2 = pos16_ref[:, 2:3]
    bi = pos16_ref[:, 3:4]
    sqi = (p0 * p0 + p1 * p1) + p2 * p2

    def dist_tile(t, _):
        c0 = pl.multiple_of(lo + t * C, 512)
        cl = pl.multiple_of(t * C, 512)
        q0 = posT_ref[0:1, pl.ds(c0, C)]
        q1 = posT_ref[1:2, pl.ds(c0, C)]
        q2 = posT_ref[2:3, pl.ds(c0, C)]
        bj = posT_ref[3:4, pl.ds(c0, C)]
        sqj = (q0 * q0 + q1 * q1) + q2 * q2
        dot = (p0 * q0 + p1 * q1) + p2 * q2
        d2 = (sqi + sqj) - 2.0 * dot
        d2 = jnp.where(bj != bi, jnp.inf, d2)
        dbuf[:, pl.ds(cl, C)] = d2
        return 0

    jax.lax.fori_loop(0, nt, dist_tile, 0)

    big_i = jnp.int32(2 ** 30)

    def min_tile(t, m):
        tile = dbuf[:, pl.ds(pl.multiple_of(t * C, 512), C)]
        return jnp.minimum(m, jnp.min(tile, axis=1, keepdims=True))
    m = jax.lax.fori_loop(0, nt, min_tile,
                          jnp.full((R, 1), jnp.inf, jnp.float32))

    for it in range(K):
        def fused(t, carry):
            found, pj, nxt = carry
            cl = pl.multiple_of(t * C, 512)
            tile = dbuf[:, pl.ds(cl, C)]
            ii = jax.lax.broadcasted_iota(jnp.int32, (R, C), 1) + t * C
            eq = tile == m
            tidx = jnp.min(jnp.where(eq, ii, big_i), axis=1, keepdims=True)
            is_here = (found >= big_i) & (tidx < big_i)
            sel = jnp.where(is_here, tidx, big_i)
            oh = ii == sel
            tile2 = jnp.where(oh, jnp.inf, tile)
            dbuf[:, pl.ds(cl, C)] = tile2
            nxt = jnp.minimum(nxt, jnp.min(tile2, axis=1, keepdims=True))
            win = pos16p_ref[pl.ds(pl.multiple_of(lo + t * C, 512), C), 0:8]
            pj = pj + jnp.dot(oh.astype(jnp.float32), win,
                              preferred_element_type=jnp.float32)
            return jnp.minimum(found, sel), pj, nxt

        idx, pj, m = jax.lax.fori_loop(
            0, nt, fused,
            (jnp.full((R, 1), big_i, jnp.int32),
             jnp.zeros((R, 8), jnp.float32),
             jnp.full((R, 1), jnp.inf, jnp.float32)))

        gidx = jnp.minimum(lo + jnp.where(idx >= big_i, 0, idx), N - 1)
        nbr_ref[:, it:it + 1] = gidx
        diff = pj - pos16_ref[:, 0:8]
        d2e = jnp.sum(diff * diff, axis=1, keepdims=True)
        dist = jnp.sqrt(d2e + 1e-12)
        dirv = diff * (1.0 / (dist + 1e-6))
        eattr_ref[it] = jnp.concatenate([dist, dirv[:, 0:7]], axis=1)


def _knn(pos16, posT8, pos16p, bounds, N, R, C, K):
    CMAX = ((N + C - 1) // C) * C
    NPAD = posT8.shape[1]
    return pl.pallas_call(
        functools.partial(_knn_body, R=R, C=C, K=K, N=N),
        grid_spec=pltpu.PrefetchScalarGridSpec(
            num_scalar_prefetch=1,
            grid=(N // R,),
            in_specs=[
                pl.BlockSpec((R, 16), lambda i, s: (i, 0)),
                pl.BlockSpec((8, NPAD), lambda i, s: (0, 0)),
                pl.BlockSpec((NPAD, 16), lambda i, s: (0, 0)),
            ],
            out_specs=[
                pl.BlockSpec((R, K), lambda i, s: (i, 0)),
                pl.BlockSpec((K, R, 8), lambda i, s: (0, i, 0)),
            ],
            scratch_shapes=[pltpu.VMEM((R, CMAX), jnp.float32)],
        ),
        out_shape=[
            jax.ShapeDtypeStruct((N, K), jnp.int32),
            jax.ShapeDtypeStruct((K, N, 8), jnp.float32),
        ],
    )(bounds, pos16, posT8, pos16p)


def _init_body(x_ref, Win, bin_, We1a, be1, h_ref, g_ref):
    h = jnp.dot(x_ref[...], Win[...],
                preferred_element_type=jnp.float32) + bin_[...]
    h_ref[...] = h
    g_ref[...] = jnp.dot(h, We1a[...],
                         preferred_element_type=jnp.float32) + be1[...]


def _init(x, Win, bin_, We1a, be1, N, R, H):
    D = x.shape[1]
    return pl.pallas_call(
        _init_body,
        grid=(N // R,),
        in_specs=[
            pl.BlockSpec((R, D), lambda i: (i, 0)),
            pl.BlockSpec((D, H), lambda i: (0, 0)),
            pl.BlockSpec((1, H), lambda i: (0, 0)),
            pl.BlockSpec((H, H), lambda i: (0, 0)),
            pl.BlockSpec((1, H), lambda i: (0, 0)),
        ],
        out_specs=[
            pl.BlockSpec((R, H), lambda i: (i, 0)),
            pl.BlockSpec((R, H), lambda i: (i, 0)),
        ],
        out_shape=[
            jax.ShapeDtypeStruct((N, H), jnp.float32),
            jax.ShapeDtypeStruct((N, H), jnp.float32),
        ],
    )(x, Win, bin_, We1a, be1)


def _sc_gather(table, idx_flat):
    E = idx_flat.shape[0]
    W = table.shape[1]
    win = 128
    grid = E // win
    mesh = plsc.VectorSubcoreMesh(core_axis_name="c", subcore_axis_name="s")

    @functools.partial(
        pl.kernel,
        out_type=jax.ShapeDtypeStruct((E, W), table.dtype),
        mesh=mesh)
    def k(x_hbm, i_hbm, o_hbm):
        def body(i_vmem, o_vmem):
            pltpu.sync_copy(x_hbm.at[i_vmem.at[0]], o_vmem)

        pltpu.emit_pipeline(
            body,
            grid=(grid,),
            in_specs=[pl.BlockSpec((1, win), index_map=lambda i: (0, i))],
            out_specs=[pl.BlockSpec((win, W), index_map=lambda i: (i, 0))],
            core_axis_name=("c", "s"),
            dimension_semantics=(pltpu.PARALLEL,),
        )(i_hbm, o_hbm)

    return k(table, idx_flat.reshape(1, E))


def _layer_body(h_ref, gg_ref, ea_ref, We1b, We1c8, be2, We2,
                Wh1a, Wh1b, bh1, Wh2, bh2, *args, R, K, H, has_next):
    if has_next:
        We1an, be1n, hout_ref, gout_ref = args
    else:
        (hout_ref,) = args

    h = h_ref[...]
    c = jnp.dot(h, We1b[...], preferred_element_type=jnp.float32)
    repC = jnp.repeat(c, K, axis=0)
    gg = gg_ref[...]
    eterm = jnp.dot(ea_ref[...], We1c8[...],
                    preferred_element_type=jnp.float32)

    m1 = gg + repC + eterm
    m1 = m1 * jax.nn.sigmoid(m1)
    m2 = jnp.dot(m1, We2[...], preferred_element_type=jnp.float32) + be2[...]
    m2 = m2 * jax.nn.sigmoid(m2)
    agg = jnp.sum(m2.reshape(R, K, H), axis=1)

    u = (jnp.dot(h, Wh1a[...], preferred_element_type=jnp.float32)
         + jnp.dot(agg, Wh1b[...], preferred_element_type=jnp.float32)
         + bh1[...])
    u = u * jax.nn.sigmoid(u)
    u = jnp.dot(u, Wh2[...], preferred_element_type=jnp.float32) + bh2[...]
    hn = h + u
    hout_ref[...] = hn
    if has_next:
        gout_ref[...] = jnp.dot(hn, We1an[...],
                                preferred_element_type=jnp.float32) + be1n[...]


def _layer(h, gg, eattr8, wts, nxt, N, R, K, H):
    has_next = nxt is not None
    whole = lambda shape: pl.BlockSpec(shape, lambda i: (0, 0))
    in_specs = [
        pl.BlockSpec((R, H), lambda i: (i, 0)),
        pl.BlockSpec((R * K, H), lambda i: (i, 0)),
        pl.BlockSpec((R * K, 8), lambda i: (i, 0)),
        whole((H, H)), whole((8, H)), whole((1, H)), whole((H, H)),
        whole((H, H)), whole((H, H)), whole((1, H)), whole((H, H)),
        whole((1, H)),
    ]
    operands = [h, gg, eattr8] + list(wts)
    out_specs = [pl.BlockSpec((R, H), lambda i: (i, 0))]
    out_shape = [jax.ShapeDtypeStruct((N, H), jnp.float32)]
    if has_next:
        in_specs += [whole((H, H)), whole((1, H))]
        operands += list(nxt)
        out_specs.append(pl.BlockSpec((R, H), lambda i: (i, 0)))
        out_shape.append(jax.ShapeDtypeStruct((N, H), jnp.float32))
    return pl.pallas_call(
        functools.partial(_layer_body, R=R, K=K, H=H, has_next=has_next),
        grid=(N // R,),
        in_specs=in_specs,
        out_specs=out_specs,
        out_shape=out_shape,
    )(*operands)


def _pool_body(h_ref, pos16_ref, Wo1, bo1, Wo2, bo2, Wo3T, bo3, out_ref,
               acc, cnt, ones_sc, *, R, B, nblk):
    i = pl.program_id(0)

    @pl.when(i == 0)
    def _():
        acc[...] = jnp.zeros_like(acc)
        cnt[...] = jnp.zeros_like(cnt)
        ones_sc[...] = jnp.ones_like(ones_sc)

    b = pos16_ref[:, 3:4].astype(jnp.int32)
    iota_b = jax.lax.broadcasted_iota(jnp.int32, (R, B), 1)
    oh = (b == iota_b).astype(jnp.float32)
    dn = (((0,), (0,)), ((), ()))
    acc[...] += jax.lax.dot_general(oh, h_ref[...], dn,
                                    preferred_element_type=jnp.float32)
    cnt[...] += jax.lax.dot_general(oh, ones_sc[...], dn,
                                    preferred_element_type=jnp.float32)

    @pl.when(i == nblk - 1)
    def _():
        pooled = acc[...] / jnp.maximum(cnt[...], 1.0)
        o = jnp.maximum(jnp.dot(pooled, Wo1[...],
                                preferred_element_type=jnp.float32)
                        + bo1[...], 0.0)
        o = jnp.maximum(jnp.dot(o, Wo2[...],
                                preferred_element_type=jnp.float32)
                        + bo2[...], 0.0)
        out_ref[...] = jnp.sum(o * Wo3T[...], axis=1,
                               keepdims=True) + bo3[...]


def _pool(h, pos16, Wo1, bo1, Wo2, bo2, Wo3T, bo3, N, R, B, H):
    whole = lambda shape: pl.BlockSpec(shape, lambda i: (0, 0))
    nblk = N // R
    return pl.pallas_call(
        functools.partial(_pool_body, R=R, B=B, nblk=nblk),
        grid=(nblk,),
        in_specs=[
            pl.BlockSpec((R, H), lambda i: (i, 0)),
            pl.BlockSpec((R, 16), lambda i: (i, 0)),
            whole((H, 2 * H)), whole((1, 2 * H)),
            whole((2 * H, H)), whole((1, H)),
            whole((1, H)), whole((1, 1)),
        ],
        out_specs=pl.BlockSpec((B, 1), lambda i: (0, 0)),
        out_shape=jax.ShapeDtypeStruct((B, 1), jnp.float32),
        scratch_shapes=[
            pltpu.VMEM((B, H), jnp.float32),
            pltpu.VMEM((B, 1), jnp.float32),
            pltpu.VMEM((R, 1), jnp.float32),
        ],
    )(h, pos16, Wo1, bo1, Wo2, bo2, Wo3T, bo3)


def kernel(x, pos, batch, params):
    N, D = x.shape
    H = params['W_in'].shape[1]
    K, B = _K, _B
    C = 512
    R = _pick_div(N, 80)

    batchf = batch.astype(jnp.float32)
    pos16 = jnp.concatenate(
        [pos, batchf[:, None], jnp.zeros((N, 16 - pos.shape[1] - 1),
                                         jnp.float32)], axis=1)

    NPAD = (((N + C - 1) // C) + 1) * C
    pt = jnp.concatenate([pos.T, batchf[None, :]], axis=0)
    pad = jnp.concatenate(
        [jnp.zeros((3, NPAD - N), jnp.float32),
         -jnp.ones((1, NPAD - N), jnp.float32)], axis=0)
    posT8 = jnp.concatenate(
        [jnp.concatenate([pt, pad], axis=1),
         jnp.zeros((4, NPAD), jnp.float32)], axis=0)

    pos16p = jnp.concatenate(
        [pos16, jnp.zeros((NPAD - N, 16), jnp.float32)], axis=0)

    ar = jnp.arange(B)
    seg_start = jnp.searchsorted(batch, ar, side='left').astype(jnp.int32)
    seg_end = jnp.searchsorted(batch, ar, side='right').astype(jnp.int32)
    firstb = batch[::R]
    lastb = batch[R - 1::R]
    bounds = jnp.stack([(seg_start[firstb] // C) * C, seg_end[lastb]], axis=1)

    nbr, eattr = _knn(pos16, posT8, pos16p, bounds, N, R, C, K)
    idx_flat = nbr.reshape(-1)
    eattr8 = eattr.transpose(1, 0, 2).reshape(N * K, 8)

    layers = params['layers']
    w0 = layers[0]
    b2 = lambda v: v.reshape(1, -1)
    We1a0 = w0['We1'][:H]
    be10 = b2(w0['be1'])
    h, g = _init(x, params['W_in'], b2(params['b_in']),
                 We1a0, be10, N, R, H)

    NC = 5 if N % (5 * R) == 0 else 1
    Nc = N // NC
    for li, p in enumerate(layers):
        We1c8 = jnp.concatenate(
            [p['We1'][2 * H:], jnp.zeros((8 - (p['We1'].shape[0] - 2 * H), H),
                                         jnp.float32)], axis=0)
        wts = (p['We1'][H:2 * H], We1c8, b2(p['be2']), p['We2'],
               p['Wh1'][:H], p['Wh1'][H:], b2(p['bh1']), p['Wh2'],
               b2(p['bh2']))
        nxt = None
        if li + 1 < len(layers):
            pn = layers[li + 1]
            nxt = (pn['We1'][:H], b2(pn['be1']))
        hs, gs = [], []
        for c in range(NC):
            sl = slice(c * Nc, (c + 1) * Nc)
            esl = slice(c * Nc * K, (c + 1) * Nc * K)
            gg_c = _sc_gather(g, idx_flat[esl])
            res = _layer(h[sl], gg_c, eattr8[esl], wts, nxt, Nc, R, K, H)
            if nxt is not None:
                hs.append(res[0])
                gs.append(res[1])
            else:
                hs.append(res[0])
        h = jnp.concatenate(hs, axis=0) if NC > 1 else hs[0]
        if nxt is not None:
            g = jnp.concatenate(gs, axis=0) if NC > 1 else gs[0]

    return _pool(h, pos16, params['Wo1'], b2(params['bo1']),
                 params['Wo2'], b2(params['bo2']),
                 params['Wo3'].T, b2(params['bo3']), N, R, B, H)
